# baseline (speedup 1.0000x reference)
import functools

import jax
import jax.numpy as jnp
from jax import lax
from jax.experimental import pallas as pl
from jax.experimental.pallas import tpu as pltpu
from jax.experimental.pallas import tpu_sc as plsc

_NUM_CORES = 2
_NUM_SUBCORES = 16
_NW = _NUM_CORES * _NUM_SUBCORES
_GCHUNK = 128


def _sc_gather(table, idx):
    B = idx.shape[0]
    V, D = table.shape
    bpw = B // _NW
    nchunks = bpw // _GCHUNK

    mesh = plsc.VectorSubcoreMesh(core_axis_name="c", subcore_axis_name="s")

    @functools.partial(
        pl.kernel,
        mesh=mesh,
        out_type=jax.ShapeDtypeStruct((B, D), jnp.float32),
        scratch_types=[
            pltpu.VMEM((bpw,), jnp.int32),
            pltpu.VMEM((bpw, D), jnp.float32),
            pltpu.SemaphoreType.DMA,
        ],
        compiler_params=pltpu.CompilerParams(use_tc_tiling_on_sc=False),
    )
    def k(table_hbm, idx_hbm, out_hbm, idx_v, rows_v, sem):
        wid = lax.axis_index("s") * _NUM_CORES + lax.axis_index("c")
        base = wid * bpw
        pltpu.sync_copy(idx_hbm.at[pl.ds(base, bpw)], idx_v)
        copies = []
        for c in range(nchunks):
            sl = pl.ds(c * _GCHUNK, _GCHUNK)
            copies.append(
                pltpu.async_copy(table_hbm.at[idx_v.at[sl]], rows_v.at[sl], sem)
            )
        for cp in copies:
            cp.wait()
        pltpu.sync_copy(rows_v, out_hbm.at[pl.ds(base, bpw)])

    return k(table, idx)


def _mlp_body(g_ref, w1_ref, b1_ref, w2_ref, b2_ref, emb_ref, out_ref):
    g = g_ref[...]
    h1 = jnp.maximum(g * w1_ref[...] + b1_ref[...], 0.0)
    h2 = jnp.dot(h1, w2_ref[...], preferred_element_type=jnp.float32)
    h2 = jnp.maximum(h2 + b2_ref[...], 0.0)
    out_ref[...] = jnp.concatenate([emb_ref[...], h2], axis=-1)


def kernel(title, grade, table, W1, b1, W2, b2):
    B = title.shape[0]
    D = table.shape[1]
    emb = _sc_gather(table, title.astype(jnp.int32))

    BLK = 2048
    grid = B // BLK
    out = pl.pallas_call(
        _mlp_body,
        grid=(grid,),
        in_specs=[
            pl.BlockSpec((BLK, 1), lambda i: (i, 0)),
            pl.BlockSpec((1, D), lambda i: (0, 0)),
            pl.BlockSpec((1, D), lambda i: (0, 0)),
            pl.BlockSpec((D, D), lambda i: (0, 0)),
            pl.BlockSpec((1, D), lambda i: (0, 0)),
            pl.BlockSpec((BLK, D), lambda i: (i, 0)),
        ],
        out_specs=pl.BlockSpec((BLK, 2 * D), lambda i: (i, 0)),
        out_shape=jax.ShapeDtypeStruct((B, 2 * D), jnp.float32),
    )(
        grade.reshape(B, 1),
        W1.reshape(1, D),
        b1.reshape(1, D),
        W2,
        b2.reshape(1, D),
        emb,
    )
    return out


# speedup vs baseline: 2.5669x; 2.5669x over previous
import functools

import jax
import jax.numpy as jnp
from jax import lax
from jax.experimental import pallas as pl
from jax.experimental.pallas import tpu as pltpu
from jax.experimental.pallas import tpu_sc as plsc

_NUM_CORES = 2
_NUM_SUBCORES = 16
_NW = _NUM_CORES * _NUM_SUBCORES
_LANES = 16


def _sc_gather_t(tbT, idx):
    D, V = tbT.shape
    B = idx.shape[0]
    HALF = B // 2

    mesh = plsc.VectorSubcoreMesh(core_axis_name="c", subcore_axis_name="s")

    @functools.partial(
        pl.kernel,
        mesh=mesh,
        out_type=jax.ShapeDtypeStruct((2 * D, B), jnp.float32),
        scratch_types=[
            pltpu.VMEM((V,), jnp.float32),
            pltpu.VMEM((B,), jnp.int32),
            pltpu.VMEM((HALF,), jnp.float32),
            pltpu.SemaphoreType.DMA,
        ],
        compiler_params=pltpu.CompilerParams(
            use_tc_tiling_on_sc=True, needs_layout_passes=False
        ),
    )
    def k(tbT_hbm, idx_hbm, out_hbm, row_v, idx_v, out_v, sem):
        wid = lax.axis_index("s") * _NUM_CORES + lax.axis_index("c")
        cp_row = pltpu.async_copy(tbT_hbm.at[wid], row_v, sem)
        pltpu.sync_copy(idx_hbm, idx_v)
        cp_row.wait()
        for h in range(2):

            def body(i, carry, h=h):
                iv = idx_v[pl.ds(h * HALF + i * _LANES, _LANES)]
                out_v[pl.ds(i * _LANES, _LANES)] = plsc.load_gather(row_v, [iv])
                return carry

            lax.fori_loop(0, HALF // _LANES, body, 0)
            pltpu.sync_copy(out_v, out_hbm.at[wid, pl.ds(h * HALF, HALF)])

    return k(tbT, idx)


def _mlp_t_body(alias_ref, g_ref, w1t_ref, b1_ref, w2t_ref, b2_ref, out_ref):
    del alias_ref
    g = g_ref[...]
    h1 = jnp.maximum(
        jnp.dot(w1t_ref[...], g, preferred_element_type=jnp.float32) + b1_ref[...],
        0.0,
    )
    h2 = jnp.dot(w2t_ref[...], h1, preferred_element_type=jnp.float32)
    out_ref[...] = jnp.maximum(h2 + b2_ref[...], 0.0)


def kernel(title, grade, table, W1, b1, W2, b2):
    B = title.shape[0]
    V, D = table.shape

    tbT = table.T
    outT = _sc_gather_t(tbT, title.astype(jnp.int32))

    outT = pl.pallas_call(
        _mlp_t_body,
        grid=(1,),
        in_specs=[
            pl.BlockSpec(memory_space=pl.ANY),
            pl.BlockSpec((1, B), lambda i: (0, 0)),
            pl.BlockSpec((D, 1), lambda i: (0, 0)),
            pl.BlockSpec((D, 1), lambda i: (0, 0)),
            pl.BlockSpec((D, D), lambda i: (0, 0)),
            pl.BlockSpec((D, 1), lambda i: (0, 0)),
        ],
        out_specs=pl.BlockSpec((D, B), lambda i: (1, 0)),
        out_shape=jax.ShapeDtypeStruct((2 * D, B), jnp.float32),
        input_output_aliases={0: 0},
    )(
        outT,
        grade.reshape(1, B),
        W1.reshape(D, 1),
        b1.reshape(D, 1),
        W2.T,
        b2.reshape(D, 1),
    )
    return outT.T


# speedup vs baseline: 2.8396x; 1.1062x over previous
import functools

import jax
import jax.numpy as jnp
from jax import lax
from jax.experimental import pallas as pl
from jax.experimental.pallas import tpu as pltpu
from jax.experimental.pallas import tpu_sc as plsc

_NUM_CORES = 2
_NUM_SUBCORES = 16
_NW = _NUM_CORES * _NUM_SUBCORES
_LANES = 16


def _sc_gather_t(tbT, idx):
    D, V = tbT.shape
    B = idx.shape[0]
    NCHUNK = 4
    CHUNK = B // NCHUNK

    mesh = plsc.VectorSubcoreMesh(core_axis_name="c", subcore_axis_name="s")

    @functools.partial(
        pl.kernel,
        mesh=mesh,
        out_type=jax.ShapeDtypeStruct((2 * D, B), jnp.float32),
        scratch_types=[
            pltpu.VMEM((V,), jnp.float32),
            pltpu.VMEM((B,), jnp.int32),
            pltpu.VMEM((CHUNK,), jnp.float32),
            pltpu.VMEM((CHUNK,), jnp.float32),
            pltpu.SemaphoreType.DMA,
            pltpu.SemaphoreType.DMA,
            pltpu.SemaphoreType.DMA,
        ],
        compiler_params=pltpu.CompilerParams(
            use_tc_tiling_on_sc=True, needs_layout_passes=False
        ),
    )
    def k(tbT_hbm, idx_hbm, out_hbm, row_v, idx_v, ob0, ob1, sem, os0, os1):
        wid = lax.axis_index("s") * _NUM_CORES + lax.axis_index("c")
        cp_row = pltpu.async_copy(tbT_hbm.at[wid], row_v, sem)
        pltpu.sync_copy(idx_hbm, idx_v)
        cp_row.wait()
        bufs = (ob0, ob1)
        sems = (os0, os1)
        pend = [None, None]
        for q in range(NCHUNK):
            b = q % 2
            if pend[b] is not None:
                pend[b].wait()

            @plsc.parallel_loop(0, CHUNK // _LANES, unroll=8)
            def body(i, q=q, b=b):
                iv = idx_v[pl.ds(q * CHUNK + i * _LANES, _LANES)]
                bufs[b][pl.ds(i * _LANES, _LANES)] = plsc.load_gather(row_v, [iv])

            pend[b] = pltpu.async_copy(
                bufs[b], out_hbm.at[wid, pl.ds(q * CHUNK, CHUNK)], sems[b]
            )
        for p in pend:
            p.wait()

    return k(tbT, idx)


def _mlp_t_body(alias_ref, g_ref, w1t_ref, b1_ref, w2t_ref, b2_ref, out_ref):
    del alias_ref
    g = g_ref[...]
    h1 = jnp.maximum(
        jnp.dot(w1t_ref[...], g, preferred_element_type=jnp.float32) + b1_ref[...],
        0.0,
    )
    h2 = jnp.dot(w2t_ref[...], h1, preferred_element_type=jnp.float32)
    out_ref[...] = jnp.maximum(h2 + b2_ref[...], 0.0)


def kernel(title, grade, table, W1, b1, W2, b2):
    B = title.shape[0]
    V, D = table.shape

    tbT = table.T
    outT = _sc_gather_t(tbT, title.astype(jnp.int32))

    BLK = 2048
    outT = pl.pallas_call(
        _mlp_t_body,
        grid=(B // BLK,),
        in_specs=[
            pl.BlockSpec(memory_space=pl.ANY),
            pl.BlockSpec((1, BLK), lambda i: (0, i)),
            pl.BlockSpec((D, 1), lambda i: (0, 0)),
            pl.BlockSpec((D, 1), lambda i: (0, 0)),
            pl.BlockSpec((D, D), lambda i: (0, 0)),
            pl.BlockSpec((D, 1), lambda i: (0, 0)),
        ],
        out_specs=pl.BlockSpec((D, BLK), lambda i: (1, i)),
        out_shape=jax.ShapeDtypeStruct((2 * D, B), jnp.float32),
        input_output_aliases={0: 0},
    )(
        outT,
        grade.reshape(1, B),
        W1.reshape(D, 1),
        b1.reshape(D, 1),
        W2.T,
        b2.reshape(D, 1),
    )
    return outT.T


# speedup vs baseline: 3.1022x; 1.0925x over previous
import functools

import jax
import jax.numpy as jnp
from jax import lax
from jax.experimental import pallas as pl
from jax.experimental.pallas import tpu as pltpu
from jax.experimental.pallas import tpu_sc as plsc

_NUM_CORES = 2
_NUM_SUBCORES = 16
_NW = _NUM_CORES * _NUM_SUBCORES
_LANES = 16


def _sc_gather_t(tbT, idx):
    D, V = tbT.shape
    B = idx.shape[0]
    NCHUNK = 4
    CHUNK = B // NCHUNK

    mesh = plsc.VectorSubcoreMesh(core_axis_name="c", subcore_axis_name="s")

    @functools.partial(
        pl.kernel,
        mesh=mesh,
        out_type=jax.ShapeDtypeStruct((2 * D, B), jnp.float32),
        scratch_types=[
            pltpu.VMEM((V,), jnp.float32),
            pltpu.VMEM((B,), jnp.int32),
            pltpu.VMEM((CHUNK,), jnp.float32),
            pltpu.VMEM((CHUNK,), jnp.float32),
            pltpu.SemaphoreType.DMA,
            pltpu.SemaphoreType.DMA,
            pltpu.SemaphoreType.DMA,
        ],
        compiler_params=pltpu.CompilerParams(
            use_tc_tiling_on_sc=True, needs_layout_passes=False
        ),
    )
    def k(tbT_hbm, idx_hbm, out_hbm, row_v, idx_v, ob0, ob1, sem, os0, os1):
        wid = lax.axis_index("s") * _NUM_CORES + lax.axis_index("c")
        cp_row = pltpu.async_copy(tbT_hbm.at[wid], row_v, sem)
        pltpu.sync_copy(idx_hbm, idx_v)
        cp_row.wait()
        bufs = (ob0, ob1)
        sems = (os0, os1)
        pend = [None, None]
        for q in range(NCHUNK):
            b = q % 2
            if pend[b] is not None:
                pend[b].wait()

            @plsc.parallel_loop(0, CHUNK // _LANES, unroll=8)
            def body(i, q=q, b=b):
                iv = idx_v[pl.ds(q * CHUNK + i * _LANES, _LANES)]
                bufs[b][pl.ds(i * _LANES, _LANES)] = plsc.load_gather(row_v, [iv])

            pend[b] = pltpu.async_copy(
                bufs[b], out_hbm.at[wid, pl.ds(q * CHUNK, CHUNK)], sems[b]
            )
        for p in pend:
            p.wait()

    return k(tbT, idx)


def _mlp_t_body(alias_ref, g_ref, w1t_ref, b1_ref, w2t_ref, b2_ref, out_ref):
    del alias_ref
    g = g_ref[...]
    h1 = jnp.maximum(
        jnp.dot(w1t_ref[...], g, preferred_element_type=jnp.float32) + b1_ref[...],
        0.0,
    )
    h2 = jnp.dot(w2t_ref[...], h1, preferred_element_type=jnp.float32)
    out_ref[...] = jnp.maximum(h2 + b2_ref[...], 0.0)


def kernel(title, grade, table, W1, b1, W2, b2):
    B = title.shape[0]
    V, D = table.shape

    tbT = table.T
    outT = _sc_gather_t(tbT, title.astype(jnp.int32))

    BLK = B
    outT = pl.pallas_call(
        _mlp_t_body,
        grid=(B // BLK,),
        in_specs=[
            pl.BlockSpec(memory_space=pl.ANY),
            pl.BlockSpec((1, BLK), lambda i: (0, i)),
            pl.BlockSpec((D, 1), lambda i: (0, 0)),
            pl.BlockSpec((D, 1), lambda i: (0, 0)),
            pl.BlockSpec((D, D), lambda i: (0, 0)),
            pl.BlockSpec((D, 1), lambda i: (0, 0)),
        ],
        out_specs=pl.BlockSpec((D, BLK), lambda i: (1, i)),
        out_shape=jax.ShapeDtypeStruct((2 * D, B), jnp.float32),
        input_output_aliases={0: 0},
    )(
        outT,
        grade.reshape(1, B),
        W1.reshape(D, 1),
        b1.reshape(D, 1),
        W2.T,
        b2.reshape(D, 1),
    )
    return outT.T
